# Initial kernel scaffold; baseline (speedup 1.0000x reference)
import jax
import jax.numpy as jnp
from jax import lax
from jax.experimental import pallas as pl
from jax.experimental.pallas import tpu as pltpu, tpu_sc as plsc

_N, _C, _H_IN, _W_IN = 20, 16, 24, 30
_H_OUT, _W_OUT = 49, 61
_P = _N * _C
_S_IN = _H_IN * _W_IN
_S_OUT = _H_OUT * _W_OUT
_NVEC = _S_IN // 16
_S_OUT_PAD = ((_S_OUT + 15) // 16) * 16
_NW = 32
_PLANES_PER_W = _P // _NW


def _unpool_body(x_hbm, idx_hbm, out_hbm, idx_v, val_v, out_v):
    c = lax.axis_index("c")
    s = lax.axis_index("s")
    wid = s * 2 + c

    zeros16 = jnp.zeros((16,), jnp.float32)

    def zbody(j, carry):
        out_v[pl.ds(j * 16, 16)] = zeros16
        return carry

    lax.fori_loop(0, _S_OUT_PAD // 16, zbody, 0)

    def pbody(q, carry):
        p = q * _NW + wid
        pltpu.sync_copy(x_hbm.at[p], val_v)
        pltpu.sync_copy(idx_hbm.at[p], idx_v)

        def sbody(i, inner):
            st = i * 16
            iv = idx_v[pl.ds(st, 16)]
            vv = val_v[pl.ds(st, 16)]
            plsc.store_scatter(out_v, [iv], vv)
            return inner

        lax.fori_loop(0, _NVEC, sbody, 0)

        pltpu.sync_copy(out_v.at[pl.ds(0, _S_OUT)], out_hbm.at[p])

        def rbody(i, inner):
            st = i * 16
            iv = idx_v[pl.ds(st, 16)]
            plsc.store_scatter(out_v, [iv], zeros16)
            return inner

        lax.fori_loop(0, _NVEC, rbody, 0)
        return carry

    lax.fori_loop(0, _PLANES_PER_W, pbody, 0)


@jax.jit
def kernel(x, indices):
    x_flat = x.reshape(_P, _S_IN)
    idx_flat = indices.astype(jnp.int32).reshape(_P, _S_IN)
    mesh = plsc.VectorSubcoreMesh(core_axis_name="c", subcore_axis_name="s")
    out = pl.kernel(
        _unpool_body,
        out_type=jax.ShapeDtypeStruct((_P, _S_OUT), jnp.float32),
        mesh=mesh,
        scratch_types=[
            pltpu.VMEM((_S_IN,), jnp.int32),
            pltpu.VMEM((_S_IN,), jnp.float32),
            pltpu.VMEM((_S_OUT_PAD,), jnp.float32),
        ],
    )(x_flat, idx_flat)
    return out.reshape(_N, _C, _H_OUT, _W_OUT)


# speedup vs baseline: 13.3927x; 13.3927x over previous
import jax
import jax.numpy as jnp
from jax import lax
from jax.experimental import pallas as pl
from jax.experimental.pallas import tpu as pltpu, tpu_sc as plsc

_N, _C, _H_IN, _W_IN = 20, 16, 24, 30
_H_OUT, _W_OUT = 49, 61
_P = _N * _C
_S_IN = _H_IN * _W_IN
_S_OUT = _H_OUT * _W_OUT
_NVEC = _S_IN // 16
_S_OUT_PAD = ((_S_OUT + 15) // 16) * 16
_NW = 32
_PLANES_PER_W = _P // _NW


def _unpool_body(x_hbm, idx_hbm, out_hbm, idx_v, val_v, out_v):
    c = lax.axis_index("c")
    s = lax.axis_index("s")
    wid = s * 2 + c

    zeros16 = jnp.zeros((16,), jnp.float32)

    def zbody(j, carry):
        out_v[pl.ds(j * 16, 16)] = zeros16
        return carry

    lax.fori_loop(0, _S_OUT_PAD // 16, zbody, 0)

    def pbody(q, carry):
        p = q * _NW + wid
        pltpu.sync_copy(x_hbm.at[pl.ds(p * _S_IN, _S_IN)], val_v)
        pltpu.sync_copy(idx_hbm.at[pl.ds(p * _S_IN, _S_IN)], idx_v)

        def sbody(i, inner):
            st = i * 16
            iv = idx_v[pl.ds(st, 16)]
            vv = val_v[pl.ds(st, 16)]
            plsc.store_scatter(out_v, [iv], vv)
            return inner

        lax.fori_loop(0, _NVEC, sbody, 0)

        pltpu.sync_copy(out_v, out_hbm.at[pl.ds(p * _S_OUT_PAD, _S_OUT_PAD)])

        def rbody(i, inner):
            st = i * 16
            iv = idx_v[pl.ds(st, 16)]
            plsc.store_scatter(out_v, [iv], zeros16)
            return inner

        lax.fori_loop(0, _NVEC, rbody, 0)
        return carry

    lax.fori_loop(0, _PLANES_PER_W, pbody, 0)


@jax.jit
def kernel(x, indices):
    x_flat = x.reshape(_P * _S_IN)
    idx_flat = indices.astype(jnp.int32).reshape(_P * _S_IN)
    mesh = plsc.VectorSubcoreMesh(core_axis_name="c", subcore_axis_name="s")
    out = pl.kernel(
        _unpool_body,
        out_type=jax.ShapeDtypeStruct((_P * _S_OUT_PAD,), jnp.float32),
        mesh=mesh,
        compiler_params=pltpu.CompilerParams(needs_layout_passes=False),
        scratch_types=[
            pltpu.VMEM((_S_IN,), jnp.int32),
            pltpu.VMEM((_S_IN,), jnp.float32),
            pltpu.VMEM((_S_OUT_PAD,), jnp.float32),
        ],
    )(x_flat, idx_flat)
    out = out.reshape(_P, _S_OUT_PAD)[:, :_S_OUT]
    return out.reshape(_N, _C, _H_OUT, _W_OUT)


# speedup vs baseline: 14.8800x; 1.1111x over previous
import jax
import jax.numpy as jnp
from jax import lax
from jax.experimental import pallas as pl
from jax.experimental.pallas import tpu as pltpu, tpu_sc as plsc

_N, _C, _H_IN, _W_IN = 20, 16, 24, 30
_H_OUT, _W_OUT = 49, 61
_P = _N * _C
_S_IN = _H_IN * _W_IN
_S_OUT = _H_OUT * _W_OUT
_NVEC = _S_IN // 16
_S_OUT_PAD = ((_S_OUT + 15) // 16) * 16
_NW = 32
_PLANES_PER_W = _P // _NW


def _unpool_body(x_hbm, idx_hbm, out_hbm,
                 idx_v0, idx_v1, idx_v2, val_v0, val_v1, out_v0, out_v1,
                 in_sems, out_sems):
    c = lax.axis_index("c")
    s = lax.axis_index("s")
    wid = s * 2 + c

    idx_bufs = [idx_v0, idx_v1, idx_v2]
    val_bufs = [val_v0, val_v1]
    out_bufs = [out_v0, out_v1]

    zeros16 = jnp.zeros((16,), jnp.float32)

    for ob in out_bufs:
        for j in range(_S_OUT_PAD // 16):
            ob[pl.ds(j * 16, 16)] = zeros16

    def start_in(q):
        p = q * _NW + wid
        sem = in_sems.at[q % 2]
        hi = pltpu.async_copy(
            idx_hbm.at[pl.ds(p * _S_IN, _S_IN)], idx_bufs[q % 3], sem)
        hv = pltpu.async_copy(
            x_hbm.at[pl.ds(p * _S_IN, _S_IN)], val_bufs[q % 2], sem)
        return hi, hv

    in_handles = {0: start_in(0)}
    out_handles = {}

    for q in range(_PLANES_PER_W):
        b = q % 2
        t = q % 3
        hi, hv = in_handles.pop(q)
        hi.wait()
        hv.wait()

        if q >= 2:
            out_handles.pop(q - 2).wait()
            idx_old = idx_bufs[(q - 2) % 3]
            for i in range(_NVEC):
                iv = idx_old[pl.ds(i * 16, 16)]
                plsc.store_scatter(out_bufs[b], [iv], zeros16)

        if q + 1 < _PLANES_PER_W:
            in_handles[q + 1] = start_in(q + 1)

        for i in range(_NVEC):
            iv = idx_bufs[t][pl.ds(i * 16, 16)]
            vv = val_bufs[b][pl.ds(i * 16, 16)]
            plsc.store_scatter(out_bufs[b], [iv], vv)

        p = q * _NW + wid
        out_handles[q] = pltpu.async_copy(
            out_bufs[b], out_hbm.at[pl.ds(p * _S_OUT_PAD, _S_OUT_PAD)],
            out_sems.at[b])

    for q in sorted(out_handles):
        out_handles.pop(q).wait()


@jax.jit
def kernel(x, indices):
    x_flat = x.reshape(_P * _S_IN)
    idx_flat = indices.astype(jnp.int32).reshape(_P * _S_IN)
    mesh = plsc.VectorSubcoreMesh(core_axis_name="c", subcore_axis_name="s")
    out = pl.kernel(
        _unpool_body,
        out_type=jax.ShapeDtypeStruct((_P * _S_OUT_PAD,), jnp.float32),
        mesh=mesh,
        compiler_params=pltpu.CompilerParams(needs_layout_passes=False),
        scratch_types=[
            pltpu.VMEM((_S_IN,), jnp.int32),
            pltpu.VMEM((_S_IN,), jnp.int32),
            pltpu.VMEM((_S_IN,), jnp.int32),
            pltpu.VMEM((_S_IN,), jnp.float32),
            pltpu.VMEM((_S_IN,), jnp.float32),
            pltpu.VMEM((_S_OUT_PAD,), jnp.float32),
            pltpu.VMEM((_S_OUT_PAD,), jnp.float32),
            pltpu.SemaphoreType.DMA((2,)),
            pltpu.SemaphoreType.DMA((2,)),
        ],
    )(x_flat, idx_flat)
    out = out.reshape(_P, _S_OUT_PAD)[:, :_S_OUT]
    return out.reshape(_N, _C, _H_OUT, _W_OUT)


# speedup vs baseline: 15.8161x; 1.0629x over previous
import jax
import jax.numpy as jnp
from jax import lax
from jax.experimental import pallas as pl
from jax.experimental.pallas import tpu as pltpu, tpu_sc as plsc

_N, _C, _H_IN, _W_IN = 20, 16, 24, 30
_H_OUT, _W_OUT = 49, 61
_P = _N * _C
_S_IN = _H_IN * _W_IN
_S_OUT = _H_OUT * _W_OUT
_NVEC = _S_IN // 16
_S_OUT_PAD = ((_S_OUT + 15) // 16) * 16
_NW = 32
_PLANES_PER_W = _P // _NW


def _unpool_body(x_hbm, idx_hbm, out_hbm,
                 idx_v0, idx_v1, idx_v2, idx_v3, val_v0, val_v1, val_v2,
                 out_v0, out_v1, in_sems, out_sems):
    c = lax.axis_index("c")
    s = lax.axis_index("s")
    wid = s * 2 + c

    idx_bufs = [idx_v0, idx_v1, idx_v2, idx_v3]
    val_bufs = [val_v0, val_v1, val_v2]
    out_bufs = [out_v0, out_v1]

    zeros16 = jnp.zeros((16,), jnp.float32)

    for ob in out_bufs:
        for j in range(_S_OUT_PAD // 16):
            ob[pl.ds(j * 16, 16)] = zeros16

    def start_in(q):
        p = q * _NW + wid
        sem = in_sems.at[q % 2]
        hi = pltpu.async_copy(
            idx_hbm.at[pl.ds(p * _S_IN, _S_IN)], idx_bufs[q % 4], sem)
        hv = pltpu.async_copy(
            x_hbm.at[pl.ds(p * _S_IN, _S_IN)], val_bufs[q % 3], sem)
        return hi, hv

    in_handles = {0: start_in(0), 1: start_in(1)}
    out_handles = {}

    for q in range(_PLANES_PER_W):
        b = q % 2
        hi, hv = in_handles.pop(q)
        hi.wait()
        hv.wait()

        if q >= 2:
            out_handles.pop(q - 2).wait()
            idx_old = idx_bufs[(q - 2) % 4]
            for i in range(_NVEC):
                iv = idx_old[pl.ds(i * 16, 16)]
                plsc.store_scatter(out_bufs[b], [iv], zeros16)

        if q + 2 < _PLANES_PER_W:
            in_handles[q + 2] = start_in(q + 2)

        for i in range(_NVEC):
            iv = idx_bufs[q % 4][pl.ds(i * 16, 16)]
            vv = val_bufs[q % 3][pl.ds(i * 16, 16)]
            plsc.store_scatter(out_bufs[b], [iv], vv)

        p = q * _NW + wid
        out_handles[q] = pltpu.async_copy(
            out_bufs[b], out_hbm.at[pl.ds(p * _S_OUT_PAD, _S_OUT_PAD)],
            out_sems.at[b])

    for q in sorted(out_handles):
        out_handles.pop(q).wait()


@jax.jit
def kernel(x, indices):
    x_flat = x.reshape(_P * _S_IN)
    idx_flat = indices.astype(jnp.int32).reshape(_P * _S_IN)
    mesh = plsc.VectorSubcoreMesh(core_axis_name="c", subcore_axis_name="s")
    out = pl.kernel(
        _unpool_body,
        out_type=jax.ShapeDtypeStruct((_P * _S_OUT_PAD,), jnp.float32),
        mesh=mesh,
        compiler_params=pltpu.CompilerParams(needs_layout_passes=False),
        scratch_types=[
            pltpu.VMEM((_S_IN,), jnp.int32),
            pltpu.VMEM((_S_IN,), jnp.int32),
            pltpu.VMEM((_S_IN,), jnp.int32),
            pltpu.VMEM((_S_IN,), jnp.int32),
            pltpu.VMEM((_S_IN,), jnp.float32),
            pltpu.VMEM((_S_IN,), jnp.float32),
            pltpu.VMEM((_S_IN,), jnp.float32),
            pltpu.VMEM((_S_OUT_PAD,), jnp.float32),
            pltpu.VMEM((_S_OUT_PAD,), jnp.float32),
            pltpu.SemaphoreType.DMA((2,)),
            pltpu.SemaphoreType.DMA((2,)),
        ],
    )(x_flat, idx_flat)
    out = out.reshape(_P, _S_OUT_PAD)[:, :_S_OUT]
    return out.reshape(_N, _C, _H_OUT, _W_OUT)


# speedup vs baseline: 16.3147x; 1.0315x over previous
import jax
import jax.numpy as jnp
from jax import lax
from jax.experimental import pallas as pl
from jax.experimental.pallas import tpu as pltpu, tpu_sc as plsc

_N, _C, _H_IN, _W_IN = 20, 16, 24, 30
_H_OUT, _W_OUT = 49, 61
_P = _N * _C
_S_IN = _H_IN * _W_IN
_S_OUT = _H_OUT * _W_OUT
_NVEC = _S_IN // 16
_S_OUT_PAD = ((_S_OUT + 15) // 16) * 16
_NW = 32
_PLANES_PER_W = _P // _NW


def _unpool_body(x_hbm, idx_hbm, out_hbm,
                 idx_v0, idx_v1, idx_v2, idx_v3, val_v0, val_v1, val_v2,
                 out_v0, out_v1, in_sems, out_sems):
    c = lax.axis_index("c")
    s = lax.axis_index("s")
    wid = s * 2 + c

    idx_bufs = [idx_v0, idx_v1, idx_v2, idx_v3]
    val_bufs = [val_v0, val_v1, val_v2]
    out_bufs = [out_v0, out_v1]

    zeros16 = jnp.zeros((16,), jnp.float32)

    for ob in out_bufs:
        for j in range(_S_OUT_PAD // 16):
            ob[pl.ds(j * 16, 16)] = zeros16

    def start_in(q):
        p = q * _NW + wid
        sem = in_sems.at[q % 2]
        hi = pltpu.async_copy(
            idx_hbm.at[pl.ds(p * _S_IN, _S_IN)], idx_bufs[q % 4], sem)
        hv = pltpu.async_copy(
            x_hbm.at[pl.ds(p * _S_IN, _S_IN)], val_bufs[q % 3], sem)
        return hi, hv

    in_handles = {0: start_in(0), 1: start_in(1)}
    out_handles = {}

    for q in range(_PLANES_PER_W):
        b = q % 2
        hi, hv = in_handles.pop(q)
        hi.wait()
        hv.wait()

        _D = 3
        if q >= 2:
            out_handles.pop(q - 2).wait()
            idx_old = idx_bufs[(q - 2) % 4]
            pend = [idx_old[pl.ds(i * 16, 16)] for i in range(_D)]
            for i in range(_NVEC):
                if i + _D < _NVEC:
                    pend.append(idx_old[pl.ds((i + _D) * 16, 16)])
                plsc.store_scatter(out_bufs[b], [pend.pop(0)], zeros16)

        if q + 2 < _PLANES_PER_W:
            in_handles[q + 2] = start_in(q + 2)

        idx_cur = idx_bufs[q % 4]
        val_cur = val_bufs[q % 3]
        pend = [(idx_cur[pl.ds(i * 16, 16)], val_cur[pl.ds(i * 16, 16)])
                for i in range(_D)]
        for i in range(_NVEC):
            if i + _D < _NVEC:
                pend.append((idx_cur[pl.ds((i + _D) * 16, 16)],
                             val_cur[pl.ds((i + _D) * 16, 16)]))
            iv, vv = pend.pop(0)
            plsc.store_scatter(out_bufs[b], [iv], vv)

        p = q * _NW + wid
        out_handles[q] = pltpu.async_copy(
            out_bufs[b], out_hbm.at[pl.ds(p * _S_OUT_PAD, _S_OUT_PAD)],
            out_sems.at[b])

    for q in sorted(out_handles):
        out_handles.pop(q).wait()


@jax.jit
def kernel(x, indices):
    x_flat = x.reshape(_P * _S_IN)
    idx_flat = indices.astype(jnp.int32).reshape(_P * _S_IN)
    mesh = plsc.VectorSubcoreMesh(core_axis_name="c", subcore_axis_name="s")
    out = pl.kernel(
        _unpool_body,
        out_type=jax.ShapeDtypeStruct((_P * _S_OUT_PAD,), jnp.float32),
        mesh=mesh,
        compiler_params=pltpu.CompilerParams(needs_layout_passes=False),
        scratch_types=[
            pltpu.VMEM((_S_IN,), jnp.int32),
            pltpu.VMEM((_S_IN,), jnp.int32),
            pltpu.VMEM((_S_IN,), jnp.int32),
            pltpu.VMEM((_S_IN,), jnp.int32),
            pltpu.VMEM((_S_IN,), jnp.float32),
            pltpu.VMEM((_S_IN,), jnp.float32),
            pltpu.VMEM((_S_IN,), jnp.float32),
            pltpu.VMEM((_S_OUT_PAD,), jnp.float32),
            pltpu.VMEM((_S_OUT_PAD,), jnp.float32),
            pltpu.SemaphoreType.DMA((2,)),
            pltpu.SemaphoreType.DMA((2,)),
        ],
    )(x_flat, idx_flat)
    out = out.reshape(_P, _S_OUT_PAD)[:, :_S_OUT]
    return out.reshape(_N, _C, _H_OUT, _W_OUT)


# speedup vs baseline: 17.3887x; 1.0658x over previous
import jax
import jax.numpy as jnp
from jax import lax
from jax.experimental import pallas as pl
from jax.experimental.pallas import tpu as pltpu, tpu_sc as plsc

_N, _C, _H_IN, _W_IN = 20, 16, 24, 30
_H_OUT, _W_OUT = 49, 61
_P = _N * _C
_S_IN = _H_IN * _W_IN
_NVEC = _S_IN // 16
_NW = 32
_PLANES_PER_W = _P // _NW
_DIV_MUL, _DIV_SHIFT = 4298, 18


def _split_hw(iv):
    hv = lax.shift_right_logical(iv * _DIV_MUL, _DIV_SHIFT)
    wv = iv - hv * _W_OUT
    return hv, wv


def _unpool_body(x_hbm, idx_hbm, out_hbm,
                 idx_v0, idx_v1, idx_v2, idx_v3, val_v0, val_v1, val_v2,
                 out_v0, out_v1, in_sems, out_sems):
    c = lax.axis_index("c")
    s = lax.axis_index("s")
    wid = s * 2 + c

    idx_bufs = [idx_v0, idx_v1, idx_v2, idx_v3]
    val_bufs = [val_v0, val_v1, val_v2]
    out_bufs = [out_v0, out_v1]

    zeros16 = jnp.zeros((16,), jnp.float32)
    lanes = lax.iota(jnp.int32, 16)

    for ob in out_bufs:
        for r in range(_H_OUT):
            hv = jnp.full((16,), r, jnp.int32)
            for cb in range(0, _W_OUT, 16):
                wv = lanes + cb
                if cb + 16 <= _W_OUT:
                    plsc.store_scatter(ob, [hv, wv], zeros16)
                else:
                    plsc.store_scatter(ob, [hv, wv], zeros16,
                                       mask=wv < _W_OUT)

    def start_in(q):
        p = q * _NW + wid
        sem = in_sems.at[q % 2]
        hi = pltpu.async_copy(
            idx_hbm.at[pl.ds(p * _S_IN, _S_IN)], idx_bufs[q % 4], sem)
        hv = pltpu.async_copy(
            x_hbm.at[pl.ds(p * _S_IN, _S_IN)], val_bufs[q % 3], sem)
        return hi, hv

    in_handles = {0: start_in(0), 1: start_in(1)}
    out_handles = {}

    for q in range(_PLANES_PER_W):
        b = q % 2
        hi, hv = in_handles.pop(q)
        hi.wait()
        hv.wait()

        _D = 3
        if q >= 2:
            out_handles.pop(q - 2).wait()
            idx_old = idx_bufs[(q - 2) % 4]
            pend = [idx_old[pl.ds(i * 16, 16)] for i in range(_D)]
            for i in range(_NVEC):
                if i + _D < _NVEC:
                    pend.append(idx_old[pl.ds((i + _D) * 16, 16)])
                hvv, wvv = _split_hw(pend.pop(0))
                plsc.store_scatter(out_bufs[b], [hvv, wvv], zeros16)

        if q + 2 < _PLANES_PER_W:
            in_handles[q + 2] = start_in(q + 2)

        idx_cur = idx_bufs[q % 4]
        val_cur = val_bufs[q % 3]
        pend = [(idx_cur[pl.ds(i * 16, 16)], val_cur[pl.ds(i * 16, 16)])
                for i in range(_D)]
        for i in range(_NVEC):
            if i + _D < _NVEC:
                pend.append((idx_cur[pl.ds((i + _D) * 16, 16)],
                             val_cur[pl.ds((i + _D) * 16, 16)]))
            iv, vv = pend.pop(0)
            hvv, wvv = _split_hw(iv)
            plsc.store_scatter(out_bufs[b], [hvv, wvv], vv)

        p = q * _NW + wid
        out_handles[q] = pltpu.async_copy(
            out_bufs[b], out_hbm.at[p // _C, p % _C], out_sems.at[b])

    for q in sorted(out_handles):
        out_handles.pop(q).wait()


@jax.jit
def kernel(x, indices):
    x_flat = x.reshape(_P * _S_IN)
    idx_flat = indices.astype(jnp.int32).reshape(_P * _S_IN)
    mesh = plsc.VectorSubcoreMesh(core_axis_name="c", subcore_axis_name="s")
    out = pl.kernel(
        _unpool_body,
        out_type=jax.ShapeDtypeStruct((_N, _C, _H_OUT, _W_OUT), jnp.float32),
        mesh=mesh,
        compiler_params=pltpu.CompilerParams(
            needs_layout_passes=False, use_tc_tiling_on_sc=True),
        scratch_types=[
            pltpu.VMEM((_S_IN,), jnp.int32),
            pltpu.VMEM((_S_IN,), jnp.int32),
            pltpu.VMEM((_S_IN,), jnp.int32),
            pltpu.VMEM((_S_IN,), jnp.int32),
            pltpu.VMEM((_S_IN,), jnp.float32),
            pltpu.VMEM((_S_IN,), jnp.float32),
            pltpu.VMEM((_S_IN,), jnp.float32),
            pltpu.VMEM((_H_OUT, _W_OUT), jnp.float32),
            pltpu.VMEM((_H_OUT, _W_OUT), jnp.float32),
            pltpu.SemaphoreType.DMA((2,)),
            pltpu.SemaphoreType.DMA((2,)),
        ],
    )(x_flat, idx_flat)
    return out


# speedup vs baseline: 21.1410x; 1.2158x over previous
import jax
import jax.numpy as jnp
from jax import lax
from jax.experimental import pallas as pl
from jax.experimental.pallas import tpu as pltpu, tpu_sc as plsc

_N, _C, _H_IN, _W_IN = 20, 16, 24, 30
_H_OUT, _W_OUT = 49, 61
_P = _N * _C
_S_IN = _H_IN * _W_IN
_NVEC = _S_IN // 16
_NW = 32
_PLANES_PER_W = _P // _NW
_DIV_MUL, _DIV_SHIFT = 4298, 18


def _split_hw(iv):
    hv = lax.shift_right_logical(iv * _DIV_MUL, _DIV_SHIFT)
    wv = iv - hv * _W_OUT
    return hv, wv


def _unpool_body(x_hbm, idx_hbm, out_hbm,
                 idx_v0, idx_v1, idx_v2, idx_v3, val_v0, val_v1, val_v2,
                 out_v0, out_v1, in_sems, out_sems):
    c = lax.axis_index("c")
    s = lax.axis_index("s")
    wid = s * 2 + c

    idx_bufs = [idx_v0, idx_v1, idx_v2, idx_v3]
    val_bufs = [val_v0, val_v1, val_v2]
    out_bufs = [out_v0, out_v1]

    zeros16 = jnp.zeros((16,), jnp.float32)
    lanes = lax.iota(jnp.int32, 16)

    for ob in out_bufs:
        for r in range(_H_OUT):
            hv = jnp.full((16,), r, jnp.int32)
            for cb in range(0, _W_OUT, 16):
                wv = lanes + cb
                if cb + 16 <= _W_OUT:
                    plsc.store_scatter(ob, [hv, wv], zeros16)
                else:
                    plsc.store_scatter(ob, [hv, wv], zeros16,
                                       mask=wv < _W_OUT)

    def start_in(q):
        p = q * _NW + wid
        sem = in_sems.at[q % 2]
        hi = pltpu.async_copy(
            idx_hbm.at[pl.ds(p * _S_IN, _S_IN)], idx_bufs[q % 4], sem)
        hv = pltpu.async_copy(
            x_hbm.at[pl.ds(p * _S_IN, _S_IN)], val_bufs[q % 3], sem)
        return hi, hv

    in_handles = {0: start_in(0), 1: start_in(1)}
    out_handles = {}

    for q in range(_PLANES_PER_W):
        b = q % 2
        hi, hv = in_handles.pop(q)
        hi.wait()
        hv.wait()

        _D = 3
        if q >= 2:
            out_handles.pop(q - 2).wait()
            idx_old = idx_bufs[(q - 2) % 4]
            pend = [idx_old[pl.ds(i * 16, 16)] for i in range(_D)]
            for i in range(_NVEC):
                if i + _D < _NVEC:
                    pend.append(idx_old[pl.ds((i + _D) * 16, 16)])
                hvv, wvv = _split_hw(pend.pop(0))
                plsc.store_scatter(out_bufs[b], [hvv, wvv], zeros16)

        if q + 2 < _PLANES_PER_W:
            in_handles[q + 2] = start_in(q + 2)

        idx_cur = idx_bufs[q % 4]
        val_cur = val_bufs[q % 3]
        pend = [(idx_cur[pl.ds(i * 16, 16)], val_cur[pl.ds(i * 16, 16)])
                for i in range(_D)]
        for i in range(_NVEC):
            if i + _D < _NVEC:
                pend.append((idx_cur[pl.ds((i + _D) * 16, 16)],
                             val_cur[pl.ds((i + _D) * 16, 16)]))
            iv, vv = pend.pop(0)
            hvv, wvv = _split_hw(iv)
            plsc.store_scatter(out_bufs[b], [hvv, wvv], vv)

        p = q * _NW + wid
        out_handles[q] = pltpu.async_copy(
            out_bufs[b], out_hbm.at[p // _C, :, p % _C, :], out_sems.at[b])

    for q in sorted(out_handles):
        out_handles.pop(q).wait()


@jax.jit
def kernel(x, indices):
    x_flat = x.reshape(_P * _S_IN)
    idx_flat = indices.astype(jnp.int32).reshape(_P * _S_IN)
    mesh = plsc.VectorSubcoreMesh(core_axis_name="c", subcore_axis_name="s")
    out = pl.kernel(
        _unpool_body,
        out_type=jax.ShapeDtypeStruct((_N, _H_OUT, _C, _W_OUT), jnp.float32),
        mesh=mesh,
        compiler_params=pltpu.CompilerParams(
            needs_layout_passes=False, use_tc_tiling_on_sc=True),
        scratch_types=[
            pltpu.VMEM((_S_IN,), jnp.int32),
            pltpu.VMEM((_S_IN,), jnp.int32),
            pltpu.VMEM((_S_IN,), jnp.int32),
            pltpu.VMEM((_S_IN,), jnp.int32),
            pltpu.VMEM((_S_IN,), jnp.float32),
            pltpu.VMEM((_S_IN,), jnp.float32),
            pltpu.VMEM((_S_IN,), jnp.float32),
            pltpu.VMEM((_H_OUT, _W_OUT), jnp.float32),
            pltpu.VMEM((_H_OUT, _W_OUT), jnp.float32),
            pltpu.SemaphoreType.DMA((2,)),
            pltpu.SemaphoreType.DMA((2,)),
        ],
    )(x_flat, idx_flat)
    return jnp.transpose(out, (0, 2, 1, 3))


# speedup vs baseline: 22.6856x; 1.0731x over previous
import jax
import jax.numpy as jnp
from jax import lax
from jax.experimental import pallas as pl
from jax.experimental.pallas import tpu as pltpu, tpu_sc as plsc

_N, _C, _H_IN, _W_IN = 20, 16, 24, 30
_H_OUT, _W_OUT = 49, 61
_P = _N * _C
_S_IN = _H_IN * _W_IN
_NVEC = _S_IN // 16
_NW = 32
_PLANES_PER_W = _P // _NW
_DIV_MUL, _DIV_SHIFT = 4298, 18


def _split_hw(iv):
    hv = lax.shift_right_logical(iv * _DIV_MUL, _DIV_SHIFT)
    wv = iv - hv * _W_OUT
    return hv, wv


def _unpool_body(x_hbm, idx_hbm, out_hbm,
                 idx_v0, idx_v1, idx_v2, idx_v3, val_v0, val_v1, val_v2,
                 out_v0, out_v1, in_sems, out_sems):
    c = lax.axis_index("c")
    s = lax.axis_index("s")
    wid = s * 2 + c

    idx_bufs = [idx_v0, idx_v1, idx_v2, idx_v3]
    val_bufs = [val_v0, val_v1, val_v2]
    out_bufs = [out_v0, out_v1]

    zeros16 = jnp.zeros((16,), jnp.float32)
    lanes = lax.iota(jnp.int32, 16)

    for ob in out_bufs:
        for r in range(_H_OUT):
            hv = jnp.full((16,), r, jnp.int32)
            for cb in range(0, _W_OUT, 16):
                wv = lanes + cb
                if cb + 16 <= _W_OUT:
                    plsc.store_scatter(ob, [hv, wv], zeros16)
                else:
                    plsc.store_scatter(ob, [hv, wv], zeros16,
                                       mask=wv < _W_OUT)

    def start_in(q):
        p = q * _NW + wid
        sem = in_sems.at[q % 2]
        hi = pltpu.async_copy(
            idx_hbm.at[p // _C, p % _C], idx_bufs[q % 4], sem)
        hv = pltpu.async_copy(
            x_hbm.at[p // _C, p % _C], val_bufs[q % 3], sem)
        return hi, hv

    in_handles = {0: start_in(0), 1: start_in(1)}
    out_handles = {}

    for q in range(_PLANES_PER_W):
        b = q % 2
        hi, hv = in_handles.pop(q)
        hi.wait()
        hv.wait()

        mask2 = lanes >= 2
        vecs = [(r, cb, None if cb == 0 else mask2)
                for r in range(_H_IN) for cb in (0, 14)]

        _D = 3
        if q >= 2:
            out_handles.pop(q - 2).wait()
            idx_old = idx_bufs[(q - 2) % 4]

            def load_i(k):
                r, cb, m = vecs[k]
                return idx_old[r, pl.ds(cb, 16)], m

            pend = [load_i(k) for k in range(_D)]
            for i in range(len(vecs)):
                if i + _D < len(vecs):
                    pend.append(load_i(i + _D))
                iv, m = pend.pop(0)
                hvv, wvv = _split_hw(iv)
                plsc.store_scatter(out_bufs[b], [hvv, wvv], zeros16, mask=m)

        if q + 2 < _PLANES_PER_W:
            in_handles[q + 2] = start_in(q + 2)

        idx_cur = idx_bufs[q % 4]
        val_cur = val_bufs[q % 3]

        def load_iv(k):
            r, cb, m = vecs[k]
            return idx_cur[r, pl.ds(cb, 16)], val_cur[r, pl.ds(cb, 16)], m

        pend = [load_iv(k) for k in range(_D)]
        for i in range(len(vecs)):
            if i + _D < len(vecs):
                pend.append(load_iv(i + _D))
            iv, vv, m = pend.pop(0)
            hvv, wvv = _split_hw(iv)
            plsc.store_scatter(out_bufs[b], [hvv, wvv], vv, mask=m)

        p = q * _NW + wid
        out_handles[q] = pltpu.async_copy(
            out_bufs[b], out_hbm.at[p // _C, :, p % _C, :], out_sems.at[b])

    for q in sorted(out_handles):
        out_handles.pop(q).wait()


@jax.jit
def kernel(x, indices):
    idx4 = indices.astype(jnp.int32)
    mesh = plsc.VectorSubcoreMesh(core_axis_name="c", subcore_axis_name="s")
    out = pl.kernel(
        _unpool_body,
        out_type=jax.ShapeDtypeStruct((_N, _H_OUT, _C, _W_OUT), jnp.float32),
        mesh=mesh,
        compiler_params=pltpu.CompilerParams(
            needs_layout_passes=False, use_tc_tiling_on_sc=True),
        scratch_types=[
            pltpu.VMEM((_H_IN, _W_IN), jnp.int32),
            pltpu.VMEM((_H_IN, _W_IN), jnp.int32),
            pltpu.VMEM((_H_IN, _W_IN), jnp.int32),
            pltpu.VMEM((_H_IN, _W_IN), jnp.int32),
            pltpu.VMEM((_H_IN, _W_IN), jnp.float32),
            pltpu.VMEM((_H_IN, _W_IN), jnp.float32),
            pltpu.VMEM((_H_IN, _W_IN), jnp.float32),
            pltpu.VMEM((_H_OUT, _W_OUT), jnp.float32),
            pltpu.VMEM((_H_OUT, _W_OUT), jnp.float32),
            pltpu.SemaphoreType.DMA((2,)),
            pltpu.SemaphoreType.DMA((2,)),
        ],
    )(x, idx4)
    return jnp.transpose(out, (0, 2, 1, 3))


# speedup vs baseline: 23.5220x; 1.0369x over previous
import jax
import jax.numpy as jnp
from jax import lax
from jax.experimental import pallas as pl
from jax.experimental.pallas import tpu as pltpu, tpu_sc as plsc

_N, _C, _H_IN, _W_IN = 20, 16, 24, 30
_H_OUT, _W_OUT = 49, 61
_P = _N * _C
_S_IN = _H_IN * _W_IN
_NVEC = _S_IN // 16
_NW = 32
_PLANES_PER_W = _P // _NW
_DIV_MUL, _DIV_SHIFT = 4298, 18


def _split_hw(iv):
    hv = lax.shift_right_logical(iv * _DIV_MUL, _DIV_SHIFT)
    wv = iv - hv * _W_OUT
    return hv, wv


def _unpool_body(x_hbm, idx_hbm, out_hbm,
                 idx_v0, idx_v1, idx_v2, idx_v3, val_v0, val_v1, val_v2,
                 out_v0, out_v1, in_sems, out_sems):
    c = lax.axis_index("c")
    s = lax.axis_index("s")
    wid = s * 2 + c

    idx_bufs = [idx_v0, idx_v1, idx_v2, idx_v3]
    val_bufs = [val_v0, val_v1, val_v2]
    out_bufs = [out_v0, out_v1]

    zeros16 = jnp.zeros((16,), jnp.float32)
    lanes = lax.iota(jnp.int32, 16)

    for ob in out_bufs:
        for r in range(_H_OUT):
            hv = jnp.full((16,), r, jnp.int32)
            for cb in range(0, _W_OUT, 16):
                wv = lanes + cb
                if cb + 16 <= _W_OUT:
                    plsc.store_scatter(ob, [hv, wv], zeros16)
                else:
                    plsc.store_scatter(ob, [hv, wv], zeros16,
                                       mask=wv < _W_OUT)

    def start_in(q):
        p = q * _NW + wid
        sem = in_sems.at[q % 2]
        hi = pltpu.async_copy(
            idx_hbm.at[p // _C, p % _C], idx_bufs[q % 4], sem)
        hv = pltpu.async_copy(
            x_hbm.at[p // _C, p % _C], val_bufs[q % 3], sem)
        return hi, hv

    in_handles = {0: start_in(0), 1: start_in(1)}
    out_handles = {}

    for q in range(_PLANES_PER_W):
        b = q % 2
        hi, hv = in_handles.pop(q)
        hi.wait()
        hv.wait()

        mask2 = lanes >= 2
        vecs = [(r, cb, None if cb == 0 else mask2)
                for r in range(_H_IN) for cb in (0, 14)]

        _D = 4
        if q >= 2:
            out_handles.pop(q - 2).wait()
            idx_old = idx_bufs[(q - 2) % 4]

            def load_i(k):
                r, cb, m = vecs[k]
                return idx_old[r, pl.ds(cb, 16)], m

            pend = [load_i(k) for k in range(_D)]
            for i in range(len(vecs)):
                if i + _D < len(vecs):
                    pend.append(load_i(i + _D))
                iv, m = pend.pop(0)
                hvv, wvv = _split_hw(iv)
                plsc.store_scatter(out_bufs[b], [hvv, wvv], zeros16, mask=m)

        if q + 2 < _PLANES_PER_W:
            in_handles[q + 2] = start_in(q + 2)

        idx_cur = idx_bufs[q % 4]
        val_cur = val_bufs[q % 3]

        def load_iv(k):
            r, cb, m = vecs[k]
            return idx_cur[r, pl.ds(cb, 16)], val_cur[r, pl.ds(cb, 16)], m

        pend = [load_iv(k) for k in range(_D)]
        for i in range(len(vecs)):
            if i + _D < len(vecs):
                pend.append(load_iv(i + _D))
            iv, vv, m = pend.pop(0)
            hvv, wvv = _split_hw(iv)
            plsc.store_scatter(out_bufs[b], [hvv, wvv], vv, mask=m)

        p = q * _NW + wid
        out_handles[q] = pltpu.async_copy(
            out_bufs[b], out_hbm.at[p // _C, :, p % _C, :], out_sems.at[b])

    for q in sorted(out_handles):
        out_handles.pop(q).wait()


@jax.jit
def kernel(x, indices):
    idx4 = indices.astype(jnp.int32)
    mesh = plsc.VectorSubcoreMesh(core_axis_name="c", subcore_axis_name="s")
    out = pl.kernel(
        _unpool_body,
        out_type=jax.ShapeDtypeStruct((_N, _H_OUT, _C, _W_OUT), jnp.float32),
        mesh=mesh,
        compiler_params=pltpu.CompilerParams(
            needs_layout_passes=False, use_tc_tiling_on_sc=True),
        scratch_types=[
            pltpu.VMEM((_H_IN, _W_IN), jnp.int32),
            pltpu.VMEM((_H_IN, _W_IN), jnp.int32),
            pltpu.VMEM((_H_IN, _W_IN), jnp.int32),
            pltpu.VMEM((_H_IN, _W_IN), jnp.int32),
            pltpu.VMEM((_H_IN, _W_IN), jnp.float32),
            pltpu.VMEM((_H_IN, _W_IN), jnp.float32),
            pltpu.VMEM((_H_IN, _W_IN), jnp.float32),
            pltpu.VMEM((_H_OUT, _W_OUT), jnp.float32),
            pltpu.VMEM((_H_OUT, _W_OUT), jnp.float32),
            pltpu.SemaphoreType.DMA((2,)),
            pltpu.SemaphoreType.DMA((2,)),
        ],
    )(x, idx4)
    return jnp.transpose(out, (0, 2, 1, 3))


# speedup vs baseline: 24.4097x; 1.0377x over previous
import jax
import jax.numpy as jnp
from jax import lax
from jax.experimental import pallas as pl
from jax.experimental.pallas import tpu as pltpu, tpu_sc as plsc

_N, _C, _H_IN, _W_IN = 20, 16, 24, 30
_H_OUT, _W_OUT = 49, 61
_P = _N * _C
_S_IN = _H_IN * _W_IN
_NVEC = _S_IN // 16
_NW = 32
_PLANES_PER_W = _P // _NW
_DIV_MUL, _DIV_SHIFT = 4298, 18


def _split_hw(iv):
    hv = lax.shift_right_logical(iv * _DIV_MUL, _DIV_SHIFT)
    wv = iv - hv * _W_OUT
    return hv, wv


def _unpool_body(x_hbm, idx_hbm, out_hbm,
                 idx_v0, idx_v1, idx_v2, idx_v3, val_v0, val_v1, val_v2,
                 out_v0, out_v1, in_sems, out_sems):
    c = lax.axis_index("c")
    s = lax.axis_index("s")
    wid = s * 2 + c

    idx_bufs = [idx_v0, idx_v1, idx_v2, idx_v3]
    val_bufs = [val_v0, val_v1, val_v2]
    out_bufs = [out_v0, out_v1]

    zeros16 = jnp.zeros((16,), jnp.float32)
    lanes = lax.iota(jnp.int32, 16)

    for ob in out_bufs:
        for r in range(_H_OUT):
            hv = jnp.full((16,), r, jnp.int32)
            for cb in range(0, _W_OUT, 16):
                wv = lanes + cb
                if cb + 16 <= _W_OUT:
                    plsc.store_scatter(ob, [hv, wv], zeros16)
                else:
                    plsc.store_scatter(ob, [hv, wv], zeros16,
                                       mask=wv < _W_OUT)

    def start_in(q):
        p = q * _NW + wid
        sem = in_sems.at[q % 2]
        hi = pltpu.async_copy(
            idx_hbm.at[p // _C, p % _C], idx_bufs[q % 4], sem)
        hv = pltpu.async_copy(
            x_hbm.at[p // _C, p % _C], val_bufs[q % 3], sem)
        return hi, hv

    in_handles = {0: start_in(0), 1: start_in(1)}
    out_handles = {}

    for q in range(_PLANES_PER_W):
        b = q % 2
        hi, hv = in_handles.pop(q)
        hi.wait()
        hv.wait()

        mask2 = lanes >= 2
        vecs = [(r, cb, None if cb == 0 else mask2)
                for r in range(_H_IN) for cb in (0, 14)]

        _D = 6
        if q >= 2:
            out_handles.pop(q - 2).wait()
            idx_old = idx_bufs[(q - 2) % 4]

            def load_i(k):
                r, cb, m = vecs[k]
                return idx_old[r, pl.ds(cb, 16)], m

            pend = [load_i(k) for k in range(_D)]
            for i in range(len(vecs)):
                if i + _D < len(vecs):
                    pend.append(load_i(i + _D))
                iv, m = pend.pop(0)
                hvv, wvv = _split_hw(iv)
                plsc.store_scatter(out_bufs[b], [hvv, wvv], zeros16, mask=m)

        if q + 2 < _PLANES_PER_W:
            in_handles[q + 2] = start_in(q + 2)

        idx_cur = idx_bufs[q % 4]
        val_cur = val_bufs[q % 3]

        def load_iv(k):
            r, cb, m = vecs[k]
            return idx_cur[r, pl.ds(cb, 16)], val_cur[r, pl.ds(cb, 16)], m

        pend = [load_iv(k) for k in range(_D)]
        for i in range(len(vecs)):
            if i + _D < len(vecs):
                pend.append(load_iv(i + _D))
            iv, vv, m = pend.pop(0)
            hvv, wvv = _split_hw(iv)
            plsc.store_scatter(out_bufs[b], [hvv, wvv], vv, mask=m)

        p = q * _NW + wid
        out_handles[q] = pltpu.async_copy(
            out_bufs[b], out_hbm.at[p // _C, :, p % _C, :], out_sems.at[b])

    for q in sorted(out_handles):
        out_handles.pop(q).wait()


@jax.jit
def kernel(x, indices):
    idx4 = indices.astype(jnp.int32)
    mesh = plsc.VectorSubcoreMesh(core_axis_name="c", subcore_axis_name="s")
    out = pl.kernel(
        _unpool_body,
        out_type=jax.ShapeDtypeStruct((_N, _H_OUT, _C, _W_OUT), jnp.float32),
        mesh=mesh,
        compiler_params=pltpu.CompilerParams(
            needs_layout_passes=False, use_tc_tiling_on_sc=True),
        scratch_types=[
            pltpu.VMEM((_H_IN, _W_IN), jnp.int32),
            pltpu.VMEM((_H_IN, _W_IN), jnp.int32),
            pltpu.VMEM((_H_IN, _W_IN), jnp.int32),
            pltpu.VMEM((_H_IN, _W_IN), jnp.int32),
            pltpu.VMEM((_H_IN, _W_IN), jnp.float32),
            pltpu.VMEM((_H_IN, _W_IN), jnp.float32),
            pltpu.VMEM((_H_IN, _W_IN), jnp.float32),
            pltpu.VMEM((_H_OUT, _W_OUT), jnp.float32),
            pltpu.VMEM((_H_OUT, _W_OUT), jnp.float32),
            pltpu.SemaphoreType.DMA((2,)),
            pltpu.SemaphoreType.DMA((2,)),
        ],
    )(x, idx4)
    return jnp.transpose(out, (0, 2, 1, 3))
